# Initial kernel scaffold; baseline (speedup 1.0000x reference)
import functools

import jax
import jax.numpy as jnp
from jax import lax
from jax.experimental import pallas as pl
from jax.experimental.pallas import tpu as pltpu
from jax.experimental.pallas import tpu_sc as plsc

N_NODES = 10000
N_EDGES = 320000
D = 128

NW = 32
G = 128
NPAD = 10240
E_PAD = 327680
EPT = E_PAD // NW
RPT = NPAD // 16
NBLK = NPAD // 128

_mesh = plsc.VectorSubcoreMesh(core_axis_name="c", subcore_axis_name="s")


@functools.partial(
    pl.kernel,
    out_type=jax.ShapeDtypeStruct((NW, NPAD), jnp.float32),
    mesh=_mesh,
    scratch_types=[
        pltpu.VMEM((EPT,), jnp.int32),
        pltpu.VMEM((NPAD,), jnp.float32),
    ],
)
def _hist(dst_hbm, hist_hbm, dst_v, hist_v):
    cid = lax.axis_index("c")
    sid = lax.axis_index("s")
    wid = sid * 2 + cid
    pltpu.sync_copy(dst_hbm.at[pl.ds(wid * EPT, EPT)], dst_v)

    zeros = jnp.zeros((16,), jnp.float32)

    @pl.loop(0, NPAD, step=16)
    def _(i):
        hist_v[pl.ds(i, 16)] = zeros

    ones = jnp.ones((16,), jnp.float32)

    @pl.loop(0, EPT, step=16)
    def _(i):
        idx = dst_v[pl.ds(i, 16)]
        plsc.addupdate_scatter(hist_v, [idx], ones)

    pltpu.sync_copy(hist_v, hist_hbm.at[wid])


@functools.partial(
    pl.kernel,
    out_type=jax.ShapeDtypeStruct((2, NPAD, D), jnp.float32),
    mesh=_mesh,
    scratch_types=[
        pltpu.VMEM((G,), jnp.int32),
        pltpu.VMEM((G,), jnp.int32),
        pltpu.VMEM((G, D), jnp.float32),
        pltpu.VMEM_SHARED((NPAD, D), jnp.float32),
    ],
)
def _agg(g_hbm, src_hbm, dst_hbm, out_hbm, src_v, dst_v, rows_v, acc_sh):
    cid = lax.axis_index("c")
    sid = lax.axis_index("s")
    wid = sid * 2 + cid

    zeros = jnp.zeros((16,), jnp.float32)

    @pl.loop(0, G)
    def _(r):
        @pl.loop(0, D, step=16)
        def _(j):
            rows_v[r, pl.ds(j, 16)] = zeros

    @pl.loop(0, RPT, step=G)
    def _(r):
        pltpu.sync_copy(rows_v, acc_sh.at[pl.ds(sid * RPT + r, G)])

    plsc.subcore_barrier()

    @pl.loop(0, EPT, step=G)
    def _(e):
        base = wid * EPT + e
        pltpu.sync_copy(src_hbm.at[pl.ds(base, G)], src_v)
        pltpu.sync_copy(dst_hbm.at[pl.ds(base, G)], dst_v)
        pltpu.sync_copy(g_hbm.at[src_v], rows_v)
        pltpu.sync_copy(rows_v, acc_sh.at[dst_v], add=True)

    plsc.subcore_barrier()

    pltpu.sync_copy(
        acc_sh.at[pl.ds(sid * RPT, RPT)],
        out_hbm.at[cid].at[pl.ds(sid * RPT, RPT)],
    )


def _dinv_col(hist_blk):
    ones = jnp.ones((NW, 1), jnp.float32)
    deg = 1.0 + lax.dot_general(
        hist_blk, ones, (((0,), (0,)), ((), ())),
        preferred_element_type=jnp.float32,
    )
    return lax.rsqrt(deg)


def _tc1_body(hist_ref, x_ref, w_ref, g_ref):
    dinv = _dinv_col(hist_ref[...])
    h = jnp.dot(x_ref[...], w_ref[...], preferred_element_type=jnp.float32)
    g_ref[...] = dinv * h


def _tc2_body(hist_ref, a_ref, g1_ref, b_ref, w_ref, g2_ref):
    dinv = _dinv_col(hist_ref[...])
    agg = a_ref[0] + a_ref[1] + g1_ref[...]
    z = jnp.maximum(dinv * agg + b_ref[...], 0.0)
    g2_ref[...] = dinv * jnp.dot(
        z, w_ref[...], preferred_element_type=jnp.float32
    )


def _tc3_body(hist_ref, a_ref, g2_ref, b_ref, o_ref):
    dinv = _dinv_col(hist_ref[...])
    agg = a_ref[0] + a_ref[1] + g2_ref[...]
    o_ref[...] = jax.nn.sigmoid(dinv * agg + b_ref[...])


def _tc1(hist, x_pad, W1):
    return pl.pallas_call(
        _tc1_body,
        grid=(NBLK,),
        in_specs=[
            pl.BlockSpec((NW, 128), lambda i: (0, i)),
            pl.BlockSpec((128, D), lambda i: (i, 0)),
            pl.BlockSpec((D, D), lambda i: (0, 0)),
        ],
        out_specs=pl.BlockSpec((128, D), lambda i: (i, 0)),
        out_shape=jax.ShapeDtypeStruct((NPAD, D), jnp.float32),
    )(hist, x_pad, W1)


def _tc2(hist, agg1, g1, b1, W2):
    return pl.pallas_call(
        _tc2_body,
        grid=(NBLK,),
        in_specs=[
            pl.BlockSpec((NW, 128), lambda i: (0, i)),
            pl.BlockSpec((2, 128, D), lambda i: (0, i, 0)),
            pl.BlockSpec((128, D), lambda i: (i, 0)),
            pl.BlockSpec((1, D), lambda i: (0, 0)),
            pl.BlockSpec((D, D), lambda i: (0, 0)),
        ],
        out_specs=pl.BlockSpec((128, D), lambda i: (i, 0)),
        out_shape=jax.ShapeDtypeStruct((NPAD, D), jnp.float32),
    )(hist, agg1, g1, b1, W2)


def _tc3(hist, agg2, g2, b2):
    return pl.pallas_call(
        _tc3_body,
        grid=(NBLK,),
        in_specs=[
            pl.BlockSpec((NW, 128), lambda i: (0, i)),
            pl.BlockSpec((2, 128, D), lambda i: (0, i, 0)),
            pl.BlockSpec((128, D), lambda i: (i, 0)),
            pl.BlockSpec((1, D), lambda i: (0, 0)),
        ],
        out_specs=pl.BlockSpec((128, D), lambda i: (i, 0)),
        out_shape=jax.ShapeDtypeStruct((NPAD, D), jnp.float32),
    )(hist, agg2, g2, b2)


@jax.jit
def kernel(x, edge_index, W1, b1, W2, b2):
    n = x.shape[0]
    src = edge_index[0].astype(jnp.int32)
    dst = edge_index[1].astype(jnp.int32)
    fill = jnp.full((E_PAD - src.shape[0],), n, jnp.int32)
    srcp = jnp.concatenate([src, fill])
    dstp = jnp.concatenate([dst, fill])
    x_pad = jnp.pad(x, ((0, NPAD - n), (0, 0)))
    b1r = b1.reshape(1, D)
    b2r = b2.reshape(1, D)

    hist = _hist(dstp)
    g1 = _tc1(hist, x_pad, W1)
    agg1 = _agg(g1, srcp, dstp)
    g2 = _tc2(hist, agg1, g1, b1r, W2)
    agg2 = _agg(g2, srcp, dstp)
    out = _tc3(hist, agg2, g2, b2r)
    return out[:n]


# speedup vs baseline: 7.4390x; 7.4390x over previous
import dataclasses
import functools

import jax
import jax.numpy as jnp
from jax import lax
from jax.experimental import pallas as pl
from jax.experimental.pallas import tpu as pltpu
from jax.experimental.pallas import tpu_sc as plsc

N_NODES = 10000
N_EDGES = 320000
D = 128

NW = 32
G = 128
NPAD = 10240
E_PAD = 327680
EPT = E_PAD // NW
RPT = NPAD // 16
NBLK = NPAD // 128

_mesh = plsc.VectorSubcoreMesh(core_axis_name="c", subcore_axis_name="s")

_sc_params = pltpu.CompilerParams()
if "needs_layout_passes" in pltpu.CompilerParams.__dataclass_fields__:
    _sc_params = dataclasses.replace(_sc_params, needs_layout_passes=False)


@functools.partial(
    pl.kernel,
    out_type=jax.ShapeDtypeStruct((NW, NPAD), jnp.float32),
    mesh=_mesh,
    scratch_types=[
        pltpu.VMEM((EPT,), jnp.int32),
        pltpu.VMEM((NPAD,), jnp.float32),
    ],
    compiler_params=_sc_params,
)
def _hist(dst_hbm, hist_hbm, dst_v, hist_v):
    cid = lax.axis_index("c")
    sid = lax.axis_index("s")
    wid = sid * 2 + cid
    pltpu.sync_copy(dst_hbm.at[pl.ds(wid * EPT, EPT)], dst_v)

    zeros = jnp.zeros((16,), jnp.float32)

    @pl.loop(0, NPAD, step=16)
    def _(i):
        hist_v[pl.ds(i, 16)] = zeros

    ones = jnp.ones((16,), jnp.float32)

    @pl.loop(0, EPT, step=16)
    def _(i):
        idx = dst_v[pl.ds(i, 16)]
        plsc.addupdate_scatter(hist_v, [idx], ones)

    pltpu.sync_copy(hist_v, hist_hbm.at[wid])


@functools.partial(
    pl.kernel,
    out_type=jax.ShapeDtypeStruct((2, NPAD, D), jnp.float32),
    mesh=_mesh,
    scratch_types=[
        pltpu.VMEM((G,), jnp.int32),
        pltpu.VMEM((G,), jnp.int32),
        pltpu.VMEM((G, D), jnp.float32),
        pltpu.VMEM_SHARED((NPAD, D), jnp.float32),
    ],
    compiler_params=_sc_params,
)
def _agg(g_hbm, src_hbm, dst_hbm, out_hbm, src_v, dst_v, rows_v, acc_sh):
    cid = lax.axis_index("c")
    sid = lax.axis_index("s")
    wid = sid * 2 + cid

    zeros = jnp.zeros((16,), jnp.float32)

    @pl.loop(0, G)
    def _(r):
        @pl.loop(0, D, step=16)
        def _(j):
            rows_v[r, pl.ds(j, 16)] = zeros

    @pl.loop(0, RPT, step=G)
    def _(r):
        pltpu.sync_copy(rows_v, acc_sh.at[pl.ds(sid * RPT + r, G)])

    plsc.subcore_barrier()

    @pl.loop(0, EPT, step=G)
    def _(e):
        base = wid * EPT + e
        pltpu.sync_copy(src_hbm.at[pl.ds(base, G)], src_v)
        pltpu.sync_copy(dst_hbm.at[pl.ds(base, G)], dst_v)
        pltpu.sync_copy(g_hbm.at[src_v], rows_v)
        pltpu.sync_copy(rows_v, acc_sh.at[dst_v], add=True)

    plsc.subcore_barrier()

    pltpu.sync_copy(
        acc_sh.at[pl.ds(sid * RPT, RPT)],
        out_hbm.at[cid].at[pl.ds(sid * RPT, RPT)],
    )


def _dinv_col(hist_blk):
    ones = jnp.ones((NW, 1), jnp.float32)
    deg = 1.0 + lax.dot_general(
        hist_blk, ones, (((0,), (0,)), ((), ())),
        preferred_element_type=jnp.float32,
    )
    return lax.rsqrt(deg)


def _tc1_body(hist_ref, x_ref, w_ref, g_ref):
    dinv = _dinv_col(hist_ref[...])
    h = jnp.dot(x_ref[...], w_ref[...], preferred_element_type=jnp.float32)
    g_ref[...] = dinv * h


def _tc2_body(hist_ref, a_ref, g1_ref, b_ref, w_ref, g2_ref):
    dinv = _dinv_col(hist_ref[...])
    agg = a_ref[0] + a_ref[1] + g1_ref[...]
    z = jnp.maximum(dinv * agg + b_ref[...], 0.0)
    g2_ref[...] = dinv * jnp.dot(
        z, w_ref[...], preferred_element_type=jnp.float32
    )


def _tc3_body(hist_ref, a_ref, g2_ref, b_ref, o_ref):
    dinv = _dinv_col(hist_ref[...])
    agg = a_ref[0] + a_ref[1] + g2_ref[...]
    o_ref[...] = jax.nn.sigmoid(dinv * agg + b_ref[...])


def _tc1(hist, x_pad, W1):
    return pl.pallas_call(
        _tc1_body,
        grid=(NBLK,),
        in_specs=[
            pl.BlockSpec((NW, 128), lambda i: (0, i)),
            pl.BlockSpec((128, D), lambda i: (i, 0)),
            pl.BlockSpec((D, D), lambda i: (0, 0)),
        ],
        out_specs=pl.BlockSpec((128, D), lambda i: (i, 0)),
        out_shape=jax.ShapeDtypeStruct((NPAD, D), jnp.float32),
    )(hist, x_pad, W1)


def _tc2(hist, agg1, g1, b1, W2):
    return pl.pallas_call(
        _tc2_body,
        grid=(NBLK,),
        in_specs=[
            pl.BlockSpec((NW, 128), lambda i: (0, i)),
            pl.BlockSpec((2, 128, D), lambda i: (0, i, 0)),
            pl.BlockSpec((128, D), lambda i: (i, 0)),
            pl.BlockSpec((1, D), lambda i: (0, 0)),
            pl.BlockSpec((D, D), lambda i: (0, 0)),
        ],
        out_specs=pl.BlockSpec((128, D), lambda i: (i, 0)),
        out_shape=jax.ShapeDtypeStruct((NPAD, D), jnp.float32),
    )(hist, agg1, g1, b1, W2)


def _tc3(hist, agg2, g2, b2):
    return pl.pallas_call(
        _tc3_body,
        grid=(NBLK,),
        in_specs=[
            pl.BlockSpec((NW, 128), lambda i: (0, i)),
            pl.BlockSpec((2, 128, D), lambda i: (0, i, 0)),
            pl.BlockSpec((128, D), lambda i: (i, 0)),
            pl.BlockSpec((1, D), lambda i: (0, 0)),
        ],
        out_specs=pl.BlockSpec((128, D), lambda i: (i, 0)),
        out_shape=jax.ShapeDtypeStruct((NPAD, D), jnp.float32),
    )(hist, agg2, g2, b2)


@jax.jit
def kernel(x, edge_index, W1, b1, W2, b2):
    n = x.shape[0]
    src = edge_index[0].astype(jnp.int32)
    dst = edge_index[1].astype(jnp.int32)
    fill = jnp.full((E_PAD - src.shape[0],), n, jnp.int32)
    srcp = jnp.concatenate([src, fill])
    dstp = jnp.concatenate([dst, fill])
    x_pad = jnp.pad(x, ((0, NPAD - n), (0, 0)))
    b1r = b1.reshape(1, D)
    b2r = b2.reshape(1, D)

    hist = _hist(dstp)
    g1 = _tc1(hist, x_pad, W1)
    agg1 = _agg(g1, srcp, dstp)
    g2 = _tc2(hist, agg1, g1, b1r, W2)
    agg2 = _agg(g2, srcp, dstp)
    out = _tc3(hist, agg2, g2, b2r)
    return out[:n]


# speedup vs baseline: 8.2857x; 1.1138x over previous
import dataclasses
import functools

import jax
import jax.numpy as jnp
from jax import lax
from jax.experimental import pallas as pl
from jax.experimental.pallas import tpu as pltpu
from jax.experimental.pallas import tpu_sc as plsc

N_NODES = 10000
N_EDGES = 320000
D = 128

NW = 32
G = 128
NPAD = 10240
E_PAD = 327680
EPT = E_PAD // NW
RPT = NPAD // 16
NBLK = NPAD // 128

_mesh = plsc.VectorSubcoreMesh(core_axis_name="c", subcore_axis_name="s")

_sc_params = pltpu.CompilerParams()
if "needs_layout_passes" in pltpu.CompilerParams.__dataclass_fields__:
    _sc_params = dataclasses.replace(_sc_params, needs_layout_passes=False)


@functools.partial(
    pl.kernel,
    out_type=jax.ShapeDtypeStruct((NW, NPAD), jnp.float32),
    mesh=_mesh,
    scratch_types=[
        pltpu.VMEM((EPT // G, G), jnp.int32),
        pltpu.VMEM((NPAD,), jnp.float32),
    ],
    compiler_params=_sc_params,
)
def _hist(dst_hbm, hist_hbm, dst_v, hist_v):
    cid = lax.axis_index("c")
    sid = lax.axis_index("s")
    wid = sid * 2 + cid
    pltpu.sync_copy(dst_hbm.at[wid], dst_v)

    zeros = jnp.zeros((16,), jnp.float32)

    @pl.loop(0, NPAD, step=16)
    def _(i):
        hist_v[pl.ds(i, 16)] = zeros

    ones = jnp.ones((16,), jnp.float32)

    @pl.loop(0, EPT // G)
    def _(ch):
        @pl.loop(0, G, step=16)
        def _(j):
            idx = dst_v[ch, pl.ds(j, 16)]
            plsc.addupdate_scatter(hist_v, [idx], ones)

    pltpu.sync_copy(hist_v, hist_hbm.at[wid])


NBUF = 2
NCH = EPT // G
WIN = NCH // 2


@functools.partial(
    pl.kernel,
    out_type=jax.ShapeDtypeStruct((2, NPAD, D), jnp.float32),
    mesh=_mesh,
    scratch_types=[
        pltpu.VMEM((WIN, G), jnp.int32),
        pltpu.VMEM((WIN, G), jnp.int32),
        pltpu.VMEM((G, D), jnp.float32),
        pltpu.VMEM((G, D), jnp.float32),
        pltpu.VMEM_SHARED((NPAD, D), jnp.float32),
        pltpu.SemaphoreType.DMA((NBUF,)),
    ],
    compiler_params=_sc_params,
)
def _agg(g_hbm, src_hbm, dst_hbm, out_hbm, src_v, dst_v, r0, r1,
         acc_sh, gsem):
    rows = (r0, r1)
    cid = lax.axis_index("c")
    sid = lax.axis_index("s")
    wid = sid * 2 + cid

    zeros = jnp.zeros((16,), jnp.float32)

    @pl.loop(0, G)
    def _(r):
        @pl.loop(0, D, step=16)
        def _(j):
            rows[0][r, pl.ds(j, 16)] = zeros

    @pl.loop(0, RPT, step=G)
    def _(r):
        pltpu.sync_copy(rows[0], acc_sh.at[pl.ds(sid * RPT + r, G)])

    plsc.subcore_barrier()

    def issue_gather(c, b):
        pltpu.async_copy(g_hbm.at[src_v.at[c]], rows[b], gsem.at[b])

    def wait_gather(c, b):
        pltpu.make_async_copy(g_hbm.at[src_v.at[c]], rows[b],
                              gsem.at[b]).wait()

    @pl.loop(0, 2)
    def _(h):
        pltpu.sync_copy(src_hbm.at[wid].at[pl.ds(h * WIN, WIN)], src_v)
        pltpu.sync_copy(dst_hbm.at[wid].at[pl.ds(h * WIN, WIN)], dst_v)

        @pl.loop(0, WIN, step=2)
        def _(c0):
            issue_gather(c0, 0)
            issue_gather(c0 + 1, 1)
            wait_gather(c0, 0)
            pltpu.sync_copy(rows[0], acc_sh.at[dst_v.at[c0]], add=True)
            wait_gather(c0 + 1, 1)
            pltpu.sync_copy(rows[1], acc_sh.at[dst_v.at[c0 + 1]], add=True)

    plsc.subcore_barrier()

    pltpu.sync_copy(
        acc_sh.at[pl.ds(sid * RPT, RPT)],
        out_hbm.at[cid].at[pl.ds(sid * RPT, RPT)],
    )


def _dinv_col(hist_blk):
    ones = jnp.ones((NW, 1), jnp.float32)
    deg = 1.0 + lax.dot_general(
        hist_blk, ones, (((0,), (0,)), ((), ())),
        preferred_element_type=jnp.float32,
    )
    return lax.rsqrt(deg)


def _tc1_body(hist_ref, x_ref, w_ref, g_ref):
    dinv = _dinv_col(hist_ref[...])
    h = jnp.dot(x_ref[...], w_ref[...], preferred_element_type=jnp.float32)
    g_ref[...] = dinv * h


def _tc2_body(hist_ref, a_ref, g1_ref, b_ref, w_ref, g2_ref):
    dinv = _dinv_col(hist_ref[...])
    agg = a_ref[0] + a_ref[1] + g1_ref[...]
    z = jnp.maximum(dinv * agg + b_ref[...], 0.0)
    g2_ref[...] = dinv * jnp.dot(
        z, w_ref[...], preferred_element_type=jnp.float32
    )


def _tc3_body(hist_ref, a_ref, g2_ref, b_ref, o_ref):
    dinv = _dinv_col(hist_ref[...])
    agg = a_ref[0] + a_ref[1] + g2_ref[...]
    o_ref[...] = jax.nn.sigmoid(dinv * agg + b_ref[...])


def _tc1(hist, x_pad, W1):
    return pl.pallas_call(
        _tc1_body,
        grid=(NBLK,),
        in_specs=[
            pl.BlockSpec((NW, 128), lambda i: (0, i)),
            pl.BlockSpec((128, D), lambda i: (i, 0)),
            pl.BlockSpec((D, D), lambda i: (0, 0)),
        ],
        out_specs=pl.BlockSpec((128, D), lambda i: (i, 0)),
        out_shape=jax.ShapeDtypeStruct((NPAD, D), jnp.float32),
    )(hist, x_pad, W1)


def _tc2(hist, agg1, g1, b1, W2):
    return pl.pallas_call(
        _tc2_body,
        grid=(NBLK,),
        in_specs=[
            pl.BlockSpec((NW, 128), lambda i: (0, i)),
            pl.BlockSpec((2, 128, D), lambda i: (0, i, 0)),
            pl.BlockSpec((128, D), lambda i: (i, 0)),
            pl.BlockSpec((1, D), lambda i: (0, 0)),
            pl.BlockSpec((D, D), lambda i: (0, 0)),
        ],
        out_specs=pl.BlockSpec((128, D), lambda i: (i, 0)),
        out_shape=jax.ShapeDtypeStruct((NPAD, D), jnp.float32),
    )(hist, agg1, g1, b1, W2)


def _tc3(hist, agg2, g2, b2):
    return pl.pallas_call(
        _tc3_body,
        grid=(NBLK,),
        in_specs=[
            pl.BlockSpec((NW, 128), lambda i: (0, i)),
            pl.BlockSpec((2, 128, D), lambda i: (0, i, 0)),
            pl.BlockSpec((128, D), lambda i: (i, 0)),
            pl.BlockSpec((1, D), lambda i: (0, 0)),
        ],
        out_specs=pl.BlockSpec((128, D), lambda i: (i, 0)),
        out_shape=jax.ShapeDtypeStruct((NPAD, D), jnp.float32),
    )(hist, agg2, g2, b2)


@jax.jit
def kernel(x, edge_index, W1, b1, W2, b2):
    n = x.shape[0]
    src = edge_index[0].astype(jnp.int32)
    dst = edge_index[1].astype(jnp.int32)
    fill = jnp.full((E_PAD - src.shape[0],), n, jnp.int32)
    srcp = jnp.concatenate([src, fill]).reshape(NW, NCH, G)
    dstp = jnp.concatenate([dst, fill]).reshape(NW, NCH, G)
    x_pad = jnp.pad(x, ((0, NPAD - n), (0, 0)))
    b1r = b1.reshape(1, D)
    b2r = b2.reshape(1, D)

    hist = _hist(dstp)
    g1 = _tc1(hist, x_pad, W1)
    agg1 = _agg(g1, srcp, dstp)
    g2 = _tc2(hist, agg1, g1, b1r, W2)
    agg2 = _agg(g2, srcp, dstp)
    out = _tc3(hist, agg2, g2, b2r)
    return out[:n]


# speedup vs baseline: 8.3096x; 1.0029x over previous
import dataclasses
import functools

import jax
import jax.numpy as jnp
from jax import lax
from jax.experimental import pallas as pl
from jax.experimental.pallas import tpu as pltpu
from jax.experimental.pallas import tpu_sc as plsc

N_NODES = 10000
N_EDGES = 320000
D = 128

NW = 32
G = 128
NPAD = 10240
E_PAD = 327680
EPT = E_PAD // NW
RPT = NPAD // 16
NBLK = NPAD // 128

_mesh = plsc.VectorSubcoreMesh(core_axis_name="c", subcore_axis_name="s")

_sc_params = pltpu.CompilerParams()
if "needs_layout_passes" in pltpu.CompilerParams.__dataclass_fields__:
    _sc_params = dataclasses.replace(_sc_params, needs_layout_passes=False)


@functools.partial(
    pl.kernel,
    out_type=jax.ShapeDtypeStruct((NW, NPAD), jnp.float32),
    mesh=_mesh,
    scratch_types=[
        pltpu.VMEM((EPT // G, G), jnp.int32),
        pltpu.VMEM((NPAD,), jnp.float32),
    ],
    compiler_params=_sc_params,
)
def _hist(dst_hbm, hist_hbm, dst_v, hist_v):
    cid = lax.axis_index("c")
    sid = lax.axis_index("s")
    wid = sid * 2 + cid
    pltpu.sync_copy(dst_hbm.at[wid], dst_v)

    zeros = jnp.zeros((16,), jnp.float32)

    @pl.loop(0, NPAD, step=16)
    def _(i):
        hist_v[pl.ds(i, 16)] = zeros

    ones = jnp.ones((16,), jnp.float32)

    @pl.loop(0, EPT // G)
    def _(ch):
        @pl.loop(0, G, step=16)
        def _(j):
            idx = dst_v[ch, pl.ds(j, 16)]
            plsc.addupdate_scatter(hist_v, [idx], ones)

    pltpu.sync_copy(hist_v, hist_hbm.at[wid])


NBUF = 2
NCH = EPT // G
WIN = NCH // 2


@functools.partial(
    pl.kernel,
    out_type=jax.ShapeDtypeStruct((2, NPAD, D), jnp.float32),
    mesh=_mesh,
    scratch_types=[
        pltpu.VMEM((WIN, G), jnp.int32),
        pltpu.VMEM((WIN, G), jnp.int32),
        pltpu.VMEM((G, D), jnp.float32),
        pltpu.VMEM((G, D), jnp.float32),
        pltpu.VMEM_SHARED((NPAD, D), jnp.float32),
        pltpu.SemaphoreType.DMA((NBUF,)),
        pltpu.SemaphoreType.DMA((NBUF,)),
    ],
    compiler_params=_sc_params,
)
def _agg(g_hbm, src_hbm, dst_hbm, out_hbm, src_v, dst_v, r0, r1,
         acc_sh, gsem, ssem):
    rows = (r0, r1)
    cid = lax.axis_index("c")
    sid = lax.axis_index("s")
    wid = sid * 2 + cid

    zeros = jnp.zeros((16,), jnp.float32)

    @pl.loop(0, G)
    def _(r):
        @pl.loop(0, D, step=16)
        def _(j):
            rows[0][r, pl.ds(j, 16)] = zeros

    @pl.loop(0, RPT, step=G)
    def _(r):
        pltpu.sync_copy(rows[0], acc_sh.at[pl.ds(sid * RPT + r, G)])

    plsc.subcore_barrier()

    def issue_gather(c, b):
        pltpu.async_copy(g_hbm.at[src_v.at[c]], rows[b], gsem.at[b])

    def wait_gather(c, b):
        pltpu.make_async_copy(g_hbm.at[src_v.at[c]], rows[b],
                              gsem.at[b]).wait()

    @pl.loop(0, 2)
    def _(h):
        pltpu.sync_copy(src_hbm.at[wid].at[pl.ds(h * WIN, WIN)], src_v)
        pltpu.sync_copy(dst_hbm.at[wid].at[pl.ds(h * WIN, WIN)], dst_v)

        @pl.loop(0, WIN, step=2)
        def _(c0):
            issue_gather(c0, 0)
            issue_gather(c0 + 1, 1)
            wait_gather(c0, 0)
            pltpu.async_copy(rows[0], acc_sh.at[dst_v.at[c0]], ssem.at[0],
                             add=True)
            wait_gather(c0 + 1, 1)
            pltpu.async_copy(rows[1], acc_sh.at[dst_v.at[c0 + 1]],
                             ssem.at[1], add=True)
            pltpu.make_async_copy(rows[0], acc_sh.at[dst_v.at[c0]],
                                  ssem.at[0]).wait()
            pltpu.make_async_copy(rows[1], acc_sh.at[dst_v.at[c0 + 1]],
                                  ssem.at[1]).wait()

    plsc.subcore_barrier()

    pltpu.sync_copy(
        acc_sh.at[pl.ds(sid * RPT, RPT)],
        out_hbm.at[cid].at[pl.ds(sid * RPT, RPT)],
    )


def _dinv_col(hist_blk):
    ones = jnp.ones((NW, 1), jnp.float32)
    deg = 1.0 + lax.dot_general(
        hist_blk, ones, (((0,), (0,)), ((), ())),
        preferred_element_type=jnp.float32,
    )
    return lax.rsqrt(deg)


def _tc1_body(hist_ref, x_ref, w_ref, g_ref):
    dinv = _dinv_col(hist_ref[...])
    h = jnp.dot(x_ref[...], w_ref[...], preferred_element_type=jnp.float32)
    g_ref[...] = dinv * h


def _tc2_body(hist_ref, a_ref, g1_ref, b_ref, w_ref, g2_ref):
    dinv = _dinv_col(hist_ref[...])
    agg = a_ref[0] + a_ref[1] + g1_ref[...]
    z = jnp.maximum(dinv * agg + b_ref[...], 0.0)
    g2_ref[...] = dinv * jnp.dot(
        z, w_ref[...], preferred_element_type=jnp.float32
    )


def _tc3_body(hist_ref, a_ref, g2_ref, b_ref, o_ref):
    dinv = _dinv_col(hist_ref[...])
    agg = a_ref[0] + a_ref[1] + g2_ref[...]
    o_ref[...] = jax.nn.sigmoid(dinv * agg + b_ref[...])


def _tc1(hist, x_pad, W1):
    return pl.pallas_call(
        _tc1_body,
        grid=(NBLK,),
        in_specs=[
            pl.BlockSpec((NW, 128), lambda i: (0, i)),
            pl.BlockSpec((128, D), lambda i: (i, 0)),
            pl.BlockSpec((D, D), lambda i: (0, 0)),
        ],
        out_specs=pl.BlockSpec((128, D), lambda i: (i, 0)),
        out_shape=jax.ShapeDtypeStruct((NPAD, D), jnp.float32),
    )(hist, x_pad, W1)


def _tc2(hist, agg1, g1, b1, W2):
    return pl.pallas_call(
        _tc2_body,
        grid=(NBLK,),
        in_specs=[
            pl.BlockSpec((NW, 128), lambda i: (0, i)),
            pl.BlockSpec((2, 128, D), lambda i: (0, i, 0)),
            pl.BlockSpec((128, D), lambda i: (i, 0)),
            pl.BlockSpec((1, D), lambda i: (0, 0)),
            pl.BlockSpec((D, D), lambda i: (0, 0)),
        ],
        out_specs=pl.BlockSpec((128, D), lambda i: (i, 0)),
        out_shape=jax.ShapeDtypeStruct((NPAD, D), jnp.float32),
    )(hist, agg1, g1, b1, W2)


def _tc3(hist, agg2, g2, b2):
    return pl.pallas_call(
        _tc3_body,
        grid=(NBLK,),
        in_specs=[
            pl.BlockSpec((NW, 128), lambda i: (0, i)),
            pl.BlockSpec((2, 128, D), lambda i: (0, i, 0)),
            pl.BlockSpec((128, D), lambda i: (i, 0)),
            pl.BlockSpec((1, D), lambda i: (0, 0)),
        ],
        out_specs=pl.BlockSpec((128, D), lambda i: (i, 0)),
        out_shape=jax.ShapeDtypeStruct((NPAD, D), jnp.float32),
    )(hist, agg2, g2, b2)


@jax.jit
def kernel(x, edge_index, W1, b1, W2, b2):
    n = x.shape[0]
    src = edge_index[0].astype(jnp.int32)
    dst = edge_index[1].astype(jnp.int32)
    fill = jnp.full((E_PAD - src.shape[0],), n, jnp.int32)
    srcp = jnp.concatenate([src, fill]).reshape(NW, NCH, G)
    dstp = jnp.concatenate([dst, fill]).reshape(NW, NCH, G)
    x_pad = jnp.pad(x, ((0, NPAD - n), (0, 0)))
    b1r = b1.reshape(1, D)
    b2r = b2.reshape(1, D)

    hist = _hist(dstp)
    g1 = _tc1(hist, x_pad, W1)
    agg1 = _agg(g1, srcp, dstp)
    g2 = _tc2(hist, agg1, g1, b1r, W2)
    agg2 = _agg(g2, srcp, dstp)
    out = _tc3(hist, agg2, g2, b2r)
    return out[:n]
